# baseline (speedup 1.0000x reference)
import jax
import jax.numpy as jnp
from jax import lax
from jax.experimental import pallas as pl
from jax.experimental.pallas import tpu as pltpu
from jax.experimental.pallas import tpu_sc as plsc

N = 10000
E = 320000
C = 128
K = 27
EPS = 1e-5

NC = 2
NS = 16
NW = NC * NS
ET = E // NW
CH = 80
NCH = ET // CH
SB = 400
NSB = ET // SB
RT0 = 624
RTL = N - (NS - 1) * RT0

BN = 2000
NB = N // BN


def _xw_body(x_ref, w_ref, out_ref):
    out_ref[...] = jnp.dot(x_ref[...], w_ref[0],
                           preferred_element_type=jnp.float32)


def _i0():
    return jnp.int32(0)


def _compute_xw(x, W):
    return pl.pallas_call(
        _xw_body,
        grid=(NB, K),
        in_specs=[
            pl.BlockSpec((BN, C), lambda nb, k: (nb, _i0())),
            pl.BlockSpec((1, C, C), lambda nb, k: (k, _i0(), _i0())),
        ],
        out_specs=pl.BlockSpec((BN, C), lambda nb, k: (k * jnp.int32(NB) + nb, _i0())),
        out_shape=jax.ShapeDtypeStruct((K * N, C), jnp.float32),
    )(x, W)


def _sc_body(xw_hbm, src_hbm, kidx_hbm, dst_hbm, zeros_hbm, out_hbm,
             src_c, kidx_c, gidx_v, dst_v, rows_v, feats_sp, sem):
    i32 = jnp.int32
    c = lax.axis_index("c").astype(i32)
    s = lax.axis_index("s").astype(i32)
    wid = c * i32(NS) + s
    ebase = wid * i32(ET)

    pltpu.sync_copy(dst_hbm.at[wid], dst_v)

    row0 = s * i32(RT0)
    zrows = rows_v.at[i32(0)]
    pltpu.sync_copy(zeros_hbm, zrows)
    for m in range(7):
        pltpu.sync_copy(zrows, feats_sp.at[pl.ds(row0 + i32(m * CH), CH)])

    @pl.when(s < i32(NS - 1))
    def _():
        pltpu.sync_copy(zrows.at[pl.ds(i32(0), RT0 - 7 * CH)],
                        feats_sp.at[pl.ds(row0 + i32(7 * CH), RT0 - 7 * CH)])

    @pl.when(s == i32(NS - 1))
    def _():
        pltpu.sync_copy(zrows, feats_sp.at[pl.ds(row0 + i32(7 * CH), CH)])

    def stage_body(b, carry):
        boff = ebase + b * i32(SB)
        pltpu.sync_copy(src_hbm.at[pl.ds(boff, SB)], src_c)
        pltpu.sync_copy(kidx_hbm.at[pl.ds(boff, SB)], kidx_c)

        def gidx_body(i, carry2):
            sl = pl.ds(i * i32(16), 16)
            osl = pl.ds(b * i32(SB) + i * i32(16), 16)
            gidx_v[osl] = kidx_c[sl] * i32(N) + src_c[sl]
            return carry2

        return lax.fori_loop(i32(0), i32(SB // 16), gidx_body, carry)

    lax.fori_loop(i32(0), i32(NSB), stage_body, i32(0))

    plsc.subcore_barrier()

    rows_a, rows_b = rows_v.at[i32(0)], rows_v.at[i32(1)]
    sem_ga, sem_gb = sem.at[i32(0)], sem.at[i32(1)]
    sem_sa, sem_sb = sem.at[i32(2)], sem.at[i32(3)]

    def issue_g(j, rbuf, rsem):
        pltpu.async_copy(
            xw_hbm.at[gidx_v.at[pl.ds(j * i32(CH), CH)]], rbuf, rsem)

    def wait_g(j, rbuf, rsem):
        pltpu.make_async_copy(
            xw_hbm.at[gidx_v.at[pl.ds(j * i32(CH), CH)]], rbuf, rsem).wait()

    def issue_s(j, rbuf, rsem):
        pltpu.async_copy(rbuf, feats_sp.at[dst_v.at[j]], rsem, add=True)

    def wait_s(j, rbuf, rsem):
        pltpu.make_async_copy(rbuf, feats_sp.at[dst_v.at[j]], rsem).wait()

    issue_g(i32(0), rows_a, sem_ga)
    issue_g(i32(1), rows_b, sem_gb)

    def chunk_body(h, carry):
        j0 = h * i32(2)
        wait_g(j0, rows_a, sem_ga)
        issue_s(j0, rows_a, sem_sa)
        wait_g(j0 + i32(1), rows_b, sem_gb)
        issue_s(j0 + i32(1), rows_b, sem_sb)
        wait_s(j0, rows_a, sem_sa)

        @pl.when(j0 + i32(2) < i32(NCH))
        def _():
            issue_g(j0 + i32(2), rows_a, sem_ga)

        wait_s(j0 + i32(1), rows_b, sem_sb)

        @pl.when(j0 + i32(3) < i32(NCH))
        def _():
            issue_g(j0 + i32(3), rows_b, sem_gb)

        return carry

    lax.fori_loop(i32(0), i32(NCH // 2), chunk_body, i32(0))

    wait_g(i32(NCH - 1), rows_a, sem_ga)
    pltpu.sync_copy(rows_a, feats_sp.at[dst_v.at[i32(NCH - 1)]], add=True)

    plsc.subcore_barrier()

    orow0 = c * i32(N) + row0

    @pl.when(s < i32(NS - 1))
    def _():
        pltpu.sync_copy(feats_sp.at[pl.ds(row0, RT0)],
                        out_hbm.at[pl.ds(orow0, RT0)])

    @pl.when(s == i32(NS - 1))
    def _():
        pltpu.sync_copy(feats_sp.at[pl.ds(row0, RTL)],
                        out_hbm.at[pl.ds(orow0, RTL)])


def _sc_gather_scatter(xw, src, kidx, dst3, zeros):
    f = pl.kernel(
        _sc_body,
        out_type=jax.ShapeDtypeStruct((2 * N, C), jnp.float32),
        mesh=plsc.VectorSubcoreMesh(core_axis_name="c", subcore_axis_name="s"),
        scratch_types=[
            pltpu.VMEM((SB,), jnp.int32),
            pltpu.VMEM((SB,), jnp.int32),
            pltpu.VMEM((ET,), jnp.int32),
            pltpu.VMEM((NCH, CH), jnp.int32),
            pltpu.VMEM((2, CH, C), jnp.float32),
            pltpu.VMEM_SHARED((N, C), jnp.float32),
            pltpu.SemaphoreType.DMA((4,)),
        ],
    )
    return f(xw, src, kidx, dst3, zeros)


BS = 2000
NBS = N // BS


def _bnfused_body(pa_ref, pb_ref, g_ref, b_ref, out_ref, st_ref):
    i = pl.program_id(0)

    @pl.when(i < NBS)
    def _():
        f = pa_ref[...] + pb_ref[...]
        s1 = jnp.sum(f, axis=0, keepdims=True)
        s2 = jnp.sum(f * f, axis=0, keepdims=True)
        blk = jnp.concatenate([s1, s2], axis=0)

        @pl.when(i == 0)
        def _():
            st_ref[...] = blk

        @pl.when(i > 0)
        def _():
            st_ref[...] += blk

    @pl.when(i >= NBS)
    def _():
        f = pa_ref[...] + pb_ref[...]
        mean = st_ref[0:1, :] * (1.0 / N)
        ex2 = st_ref[1:2, :] * (1.0 / N)
        var = ex2 - mean * mean
        scale = g_ref[...] * lax.rsqrt(var + EPS)
        out_ref[...] = jnp.maximum((f - mean) * scale + b_ref[...], 0.0)


def _bn(partials, gamma, beta):
    half = lambda i: jnp.where(i < NBS, i, i - jnp.int32(NBS))
    return pl.pallas_call(
        _bnfused_body,
        grid=(2 * NBS,),
        in_specs=[
            pl.BlockSpec((BS, C), lambda i: (half(i), _i0())),
            pl.BlockSpec((BS, C), lambda i: (jnp.int32(NBS) + half(i), _i0())),
            pl.BlockSpec((1, C), lambda i: (_i0(), _i0())),
            pl.BlockSpec((1, C), lambda i: (_i0(), _i0())),
        ],
        out_specs=pl.BlockSpec((BS, C), lambda i: (half(i), _i0())),
        out_shape=jax.ShapeDtypeStruct((N, C), jnp.float32),
        scratch_shapes=[pltpu.VMEM((2, C), jnp.float32)],
    )(partials, partials, gamma, beta)


def kernel(x, edge_index, kernel_idx, W, gamma, beta):
    x = x.astype(jnp.float32)
    W = W.astype(jnp.float32)
    src = edge_index[0].astype(jnp.int32)
    dst = edge_index[1].astype(jnp.int32)
    kidx = kernel_idx.astype(jnp.int32)
    dst3 = dst.reshape(NW, NCH, CH)
    zeros = jnp.zeros((CH, C), jnp.float32)

    xw = _compute_xw(x, W)
    partials = _sc_gather_scatter(xw, src, kidx, dst3, zeros)
    return _bn(partials,
               gamma.astype(jnp.float32).reshape(1, C),
               beta.astype(jnp.float32).reshape(1, C))
